# baseline (speedup 1.0000x reference)
import jax
import jax.numpy as jnp
import numpy as np
from jax.experimental import pallas as pl
from jax.experimental.pallas import tpu as pltpu

H = 256
N = 10000
NPAD = 10240
NROWS = NPAD // 128
OBS = 128
NL = 2
K = 5000
EPS = 1e-5
CH = 2048
NCH = NPAD // CH
TAIL = N - (NCH - 1) * CH
PR = CH // 128

_PREC = jax.lax.Precision.DEFAULT


def _dot(a, b, dims=((1,), (0,))):
    return jax.lax.dot_general(a, b, dimension_numbers=(dims, ((), ())),
                               preferred_element_type=jnp.float32,
                               precision=_PREC)


def _i32(v):
    return jnp.int32(np.uint32(v & 0xFFFFFFFF).view(np.int32))


def _fused_kernel(idx_ref,
                  x_ref, tf_Wv_ref, tf_bv_ref, tf_Wo_ref, tf_bo_ref,
                  ln1_g_ref, ln1_b_ref, ff_W1_ref, ff_b1_ref, ff_W2_ref,
                  ff_b2_ref, ln2_g_ref, ln2_b_ref, pool_w_ref,
                  fcq_W_ref, fcq_b_ref, fcq_g_ref, fcq_be_ref,
                  fcv_W_ref, fcv_b_ref, fcv_g_ref, fcv_be_ref,
                  bil_W_ref, bil_b_ref, gfc_W_ref, gfc_b_ref,
                  logits_ref, dec_ref, a_scr, b_scr, h_scr, sp_scr):
    f32 = jnp.float32

    def gather_body(i, carry):
        ridx = idx_ref[i]
        h_scr[pl.ds(i, 1), :] = x_ref[pl.ds(ridx, 1), :]
        return carry

    jax.lax.fori_loop(0, OBS, gather_body, 0, unroll=8)
    h = h_scr[...]

    def _ln(t, g, b):
        mu = jnp.mean(t, axis=-1, keepdims=True)
        var = jnp.mean((t - mu) * (t - mu), axis=-1, keepdims=True)
        return (t - mu) * jax.lax.rsqrt(var + EPS) * g + b

    for l in range(NL):
        v = _dot(h, tf_Wv_ref[l]) + tf_bv_ref[pl.ds(l, 1), :]
        a = _dot(v, tf_Wo_ref[l]) + tf_bo_ref[pl.ds(l, 1), :]
        h = _ln(h + a, ln1_g_ref[pl.ds(l, 1), :], ln1_b_ref[pl.ds(l, 1), :])
        ff = jnp.maximum(_dot(h, ff_W1_ref[l]) + ff_b1_ref[pl.ds(l, 1), :],
                         0.0)
        ff = _dot(ff, ff_W2_ref[l]) + ff_b2_ref[pl.ds(l, 1), :]
        h = _ln(h + ff, ln2_g_ref[pl.ds(l, 1), :], ln2_b_ref[pl.ds(l, 1), :])
    h = jnp.maximum(h, 0.0)
    sc = _dot(h, pool_w_ref[...], ((1,), (1,)))
    sc = jnp.exp(sc - jnp.max(sc, axis=0, keepdims=True))
    sc = sc / jnp.sum(sc, axis=0, keepdims=True)
    obs_k = _dot(sc, h, ((0,), (0,)))

    u0 = _dot(obs_k, bil_W_ref[0])
    u1 = _dot(obs_k, bil_W_ref[1])

    npadf = float(NPAD - N)

    def bn_affine(s1, s2, z_pad, g, be):
        s1 = s1 - npadf * z_pad
        s2 = s2 - npadf * z_pad * z_pad
        mu = s1 / float(N)
        var = s2 / float(N) - mu * mu
        scale = jax.lax.rsqrt(var + EPS) * g
        return scale, be - mu * scale

    bf16 = jnp.bfloat16

    def mlp2(dst_ref, W_ref, b_ref, g_ref, be_ref):
        W1 = W_ref[0]
        b1 = b_ref[pl.ds(0, 1), :]

        def mm1_body(c, carry):
            s1, s2 = carry
            z = _dot(x_ref[pl.ds(c * CH, CH), :].astype(bf16), W1) + b1
            dst_ref[pl.ds(c * CH, CH), :] = z
            return (s1 + jnp.sum(z, axis=0, keepdims=True),
                    s2 + jnp.sum(z * z, axis=0, keepdims=True))

        s1, s2 = jax.lax.fori_loop(
            0, NCH - 1, mm1_body,
            (jnp.zeros((1, H), f32), jnp.zeros((1, H), f32)))
        zt = _dot(x_ref[pl.ds((NCH - 1) * CH, TAIL), :].astype(bf16),
                  W1) + b1
        dst_ref[pl.ds((NCH - 1) * CH, TAIL), :] = zt
        s1 = s1 + jnp.sum(zt, axis=0, keepdims=True)
        s2 = s2 + jnp.sum(zt * zt, axis=0, keepdims=True)
        dst_ref[pl.ds(N, NPAD - N), :] = jnp.broadcast_to(b1, (NPAD - N, H))
        mu1 = s1 / float(N)
        var1 = s2 / float(N) - mu1 * mu1
        sc1 = jax.lax.rsqrt(var1 + EPS) * g_ref[pl.ds(0, 1), :]
        sh1 = be_ref[pl.ds(0, 1), :] - mu1 * sc1

        W2 = W_ref[1]
        b2 = b_ref[pl.ds(1, 1), :]

        def mm2_body(c, carry):
            s1, s2 = carry
            y = jnp.maximum(dst_ref[pl.ds(c * CH, CH), :] * sc1 + sh1, 0.0)
            z = _dot(y.astype(bf16), W2) + b2
            dst_ref[pl.ds(c * CH, CH), :] = z
            return (s1 + jnp.sum(z, axis=0, keepdims=True),
                    s2 + jnp.sum(z * z, axis=0, keepdims=True))

        s1, s2 = jax.lax.fori_loop(
            0, NCH, mm2_body,
            (jnp.zeros((1, H), f32), jnp.zeros((1, H), f32)))
        y_pad1 = jnp.maximum(b1 * sc1 + sh1, 0.0)
        z_pad2 = _dot(y_pad1.astype(bf16), W2) + b2
        return bn_affine(s1, s2, z_pad2, g_ref[pl.ds(1, 1), :],
                         be_ref[pl.ds(1, 1), :])

    scq, shq = mlp2(a_scr, fcq_W_ref, fcq_b_ref, fcq_g_ref, fcq_be_ref)

    ust = jnp.concatenate([u0, u1], axis=0)
    bb = bil_b_ref[...]
    b0 = bil_b_ref[0:1, 0:1]
    u03 = u0.reshape(1, 1, H)

    def dec_body(c, carry):
        xq = jnp.maximum(a_scr[pl.ds(c * CH, CH), :] * scq + shq, 0.0)
        dec_ref[pl.ds(c * CH, CH), :] = _dot(xq, ust, ((1,), (1,))) + bb
        sp_scr[pl.ds(c * PR, PR), :] = \
            jnp.sum(xq.reshape(PR, 128, H) * u03, axis=2) + b0
        return carry

    jax.lax.fori_loop(0, NCH, dec_body, 0)
    scorep = sp_scr[...]

    row_iota = jax.lax.broadcasted_iota(jnp.int32, (NROWS, 128), 0)
    col_iota = jax.lax.broadcasted_iota(jnp.int32, (NROWS, 128), 1)
    validp = (row_iota * 128 + col_iota) < N
    SIGN = _i32(0x80000000)
    kp = jax.lax.bitcast_convert_type(scorep, jnp.int32)
    kp = jnp.where(kp < 0, kp ^ _i32(0x7FFFFFFF), kp)
    up = jnp.where(validp, kp ^ SIGN, _i32(0))

    prefix = jnp.zeros((1, 1), jnp.int32)
    remaining = jnp.full((1, 1), K, jnp.int32)
    for bit in range(30, -2, -2):
        mask_hi = _i32(~((1 << bit) - 1))
        masked = up & mask_hi
        c3 = jnp.sum((masked == (prefix | _i32(3 << bit)))
                     .astype(jnp.int32), axis=(0, 1)).reshape(1, 1)
        c2 = jnp.sum((masked == (prefix | _i32(2 << bit)))
                     .astype(jnp.int32), axis=(0, 1)).reshape(1, 1)
        c1 = jnp.sum((masked == (prefix | _i32(1 << bit)))
                     .astype(jnp.int32), axis=(0, 1)).reshape(1, 1)
        s32 = c3
        s21 = c3 + c2
        s10 = s21 + c1
        sel = jnp.where(
            s32 >= remaining, _i32(3),
            jnp.where(s21 >= remaining, _i32(2),
                      jnp.where(s10 >= remaining, _i32(1), _i32(0))))
        prefix = prefix | (sel << bit)
        remaining = remaining - jnp.where(
            sel == 3, _i32(0),
            jnp.where(sel == 2, s32, jnp.where(sel == 1, s21, s10)))

    t_signed = prefix ^ SIGN

    selp = validp & (kp >= t_signed)
    m = jnp.max(jnp.where(selp, scorep, -jnp.inf)).reshape(1, 1)

    scv, shv = mlp2(b_scr, fcv_W_ref, fcv_b_ref, fcv_g_ref, fcv_be_ref)

    def pool_body(c, carry):
        pool, zsum = carry
        s_col = dec_ref[pl.ds(c * CH, CH), 0:1]
        kc = jax.lax.bitcast_convert_type(s_col, jnp.int32)
        kc = jnp.where(kc < 0, kc ^ _i32(0x7FFFFFFF), kc)
        niota = jax.lax.broadcasted_iota(jnp.int32, (CH, 1), 0) + c * CH
        sel = (niota < N) & (kc >= t_signed)
        w = jnp.where(sel, jnp.exp(s_col - m), 0.0)
        xv = jnp.maximum(b_scr[pl.ds(c * CH, CH), :] * scv + shv, 0.0)
        return (pool + jnp.sum(w * xv, axis=0, keepdims=True),
                zsum + jnp.sum(w, axis=0, keepdims=True))

    pool, zsum = jax.lax.fori_loop(
        0, NCH, pool_body,
        (jnp.zeros((1, H), f32), jnp.zeros((1, 1), f32)))
    pooled = pool / zsum

    logits_ref[...] = _dot(pooled, gfc_W_ref[...], ((1,), (1,))) \
        + gfc_b_ref[...]


def kernel(x, obs_x_idx, edge_index_01, edge_index_2, tf_Wv, tf_bv, tf_Wo,
           tf_bo, ln1_g, ln1_b, ff_W1, ff_b1, ff_W2, ff_b2, ln2_g, ln2_b,
           pool_w, pool_b, fcq_W, fcq_b, fcq_g, fcq_be, fcv_W, fcv_b, fcv_g,
           fcv_be, bil_W, bil_b, gfc_W, gfc_b):
    del edge_index_01, edge_index_2, pool_b
    idx = obs_x_idx.astype(jnp.int32)
    swap = lambda w: jnp.swapaxes(w, 1, 2)

    vmem = pl.BlockSpec(memory_space=pltpu.VMEM)
    bfc = lambda w: jnp.swapaxes(w, 1, 2).astype(jnp.bfloat16)
    operands = [
        x, swap(tf_Wv), tf_bv, swap(tf_Wo), tf_bo, ln1_g, ln1_b,
        swap(ff_W1), ff_b1, swap(ff_W2), ff_b2, ln2_g, ln2_b,
        pool_w.reshape(1, H),
        bfc(fcq_W), fcq_b, fcq_g, fcq_be, bfc(fcv_W), fcv_b, fcv_g, fcv_be,
        bil_W, bil_b.reshape(1, 2), gfc_W, gfc_b.reshape(1, 2),
    ]

    logits, dec = pl.pallas_call(
        _fused_kernel,
        grid_spec=pltpu.PrefetchScalarGridSpec(
            num_scalar_prefetch=1,
            grid=(),
            in_specs=[vmem] * len(operands),
            out_specs=[vmem, vmem],
            scratch_shapes=[
                pltpu.VMEM((NPAD, H), jnp.float32),
                pltpu.VMEM((NPAD, H), jnp.float32),
                pltpu.VMEM((OBS, H), jnp.float32),
                pltpu.VMEM((NROWS, 128), jnp.float32),
            ],
        ),
        out_shape=[
            jax.ShapeDtypeStruct((1, 2), jnp.float32),
            jax.ShapeDtypeStruct((NPAD, 2), jnp.float32),
        ],
    )(idx, *operands)
    return (logits, dec[:N])


# speedup vs baseline: 1.1053x; 1.1053x over previous
import jax
import jax.numpy as jnp
import numpy as np
from jax.experimental import pallas as pl
from jax.experimental.pallas import tpu as pltpu

H = 256
N = 10000
NPAD = 10240
NROWS = NPAD // 128
OBS = 128
NL = 2
K = 5000
EPS = 1e-5
CH = 2048
NCH = NPAD // CH
TAIL = N - (NCH - 1) * CH
PR = CH // 128

_PREC = jax.lax.Precision.DEFAULT


def _dot(a, b, dims=((1,), (0,))):
    return jax.lax.dot_general(a, b, dimension_numbers=(dims, ((), ())),
                               preferred_element_type=jnp.float32,
                               precision=_PREC)


def _i32(v):
    return jnp.int32(np.uint32(v & 0xFFFFFFFF).view(np.int32))


def _fused_kernel(idx_ref,
                  x_ref, tf_Wv_ref, tf_bv_ref, tf_Wo_ref, tf_bo_ref,
                  ln1_g_ref, ln1_b_ref, ff_W1_ref, ff_b1_ref, ff_W2_ref,
                  ff_b2_ref, ln2_g_ref, ln2_b_ref, pool_w_ref,
                  fcq_W_ref, fcq_b_ref, fcq_g_ref, fcq_be_ref,
                  fcv_W_ref, fcv_b_ref, fcv_g_ref, fcv_be_ref,
                  bil_W_ref, bil_b_ref, gfc_W_ref, gfc_b_ref,
                  logits_ref, dec_ref, a_scr, b_scr, h_scr, sp_scr):
    f32 = jnp.float32
    bf16 = jnp.bfloat16

    def gather_body(i, carry):
        ridx = idx_ref[i]
        h_scr[pl.ds(i, 1), :] = x_ref[pl.ds(ridx, 1), :]
        return carry

    jax.lax.fori_loop(0, OBS, gather_body, 0, unroll=8)
    h = h_scr[...]

    def _ln(t, g, b):
        mu = jnp.mean(t, axis=-1, keepdims=True)
        var = jnp.mean((t - mu) * (t - mu), axis=-1, keepdims=True)
        return (t - mu) * jax.lax.rsqrt(var + EPS) * g + b

    for l in range(NL):
        v = _dot(h, tf_Wv_ref[l]) + tf_bv_ref[pl.ds(l, 1), :]
        a = _dot(v, tf_Wo_ref[l]) + tf_bo_ref[pl.ds(l, 1), :]
        h = _ln(h + a, ln1_g_ref[pl.ds(l, 1), :], ln1_b_ref[pl.ds(l, 1), :])
        ff = jnp.maximum(_dot(h, ff_W1_ref[l]) + ff_b1_ref[pl.ds(l, 1), :],
                         0.0)
        ff = _dot(ff, ff_W2_ref[l]) + ff_b2_ref[pl.ds(l, 1), :]
        h = _ln(h + ff, ln2_g_ref[pl.ds(l, 1), :], ln2_b_ref[pl.ds(l, 1), :])
    h = jnp.maximum(h, 0.0)
    sc = _dot(h, pool_w_ref[...], ((1,), (1,)))
    sc = jnp.exp(sc - jnp.max(sc, axis=0, keepdims=True))
    sc = sc / jnp.sum(sc, axis=0, keepdims=True)
    obs_k = _dot(sc, h, ((0,), (0,)))

    u0 = _dot(obs_k, bil_W_ref[0])
    u1 = _dot(obs_k, bil_W_ref[1])

    npadf = float(NPAD - N)

    def chunk_rows(c):
        return TAIL if c == NCH - 1 else CH

    def mlp_layer1(dst_ref, W1, b1, g1, be1):
        s1 = jnp.zeros((1, H), f32)
        s2 = jnp.zeros((1, H), f32)
        for c in range(NCH):
            rows = chunk_rows(c)
            z = _dot(x_ref[pl.ds(c * CH, rows), :].astype(bf16), W1) + b1
            dst_ref[pl.ds(c * CH, rows), :] = z
            s1 = s1 + jnp.sum(z, axis=0, keepdims=True)
            s2 = s2 + jnp.sum(z * z, axis=0, keepdims=True)
        dst_ref[pl.ds(N, NPAD - N), :] = jnp.broadcast_to(b1, (NPAD - N, H))
        mu = s1 / float(N)
        var = s2 / float(N) - mu * mu
        sc1 = jax.lax.rsqrt(var + EPS) * g1
        return sc1, be1 - mu * sc1

    def mlp_layer2(dst_ref, W2, b2, g2, be2, sc1, sh1, b1):
        s1 = jnp.zeros((1, H), f32)
        s2 = jnp.zeros((1, H), f32)
        for c in range(NCH):
            y = jnp.maximum(dst_ref[pl.ds(c * CH, CH), :] * sc1 + sh1, 0.0)
            z = _dot(y.astype(bf16), W2) + b2
            dst_ref[pl.ds(c * CH, CH), :] = z
            s1 = s1 + jnp.sum(z, axis=0, keepdims=True)
            s2 = s2 + jnp.sum(z * z, axis=0, keepdims=True)
        y_pad1 = jnp.maximum(b1 * sc1 + sh1, 0.0)
        z_pad2 = _dot(y_pad1.astype(bf16), W2) + b2
        s1 = s1 - npadf * z_pad2
        s2 = s2 - npadf * z_pad2 * z_pad2
        mu = s1 / float(N)
        var = s2 / float(N) - mu * mu
        sc2 = jax.lax.rsqrt(var + EPS) * g2
        return sc2, be2 - mu * sc2

    bq1 = fcq_b_ref[pl.ds(0, 1), :]
    scq1, shq1 = mlp_layer1(a_scr, fcq_W_ref[0], bq1,
                            fcq_g_ref[pl.ds(0, 1), :],
                            fcq_be_ref[pl.ds(0, 1), :])
    scq, shq = mlp_layer2(a_scr, fcq_W_ref[1], fcq_b_ref[pl.ds(1, 1), :],
                          fcq_g_ref[pl.ds(1, 1), :],
                          fcq_be_ref[pl.ds(1, 1), :], scq1, shq1, bq1)

    ust = jnp.concatenate([u0, u1], axis=0)
    bb = bil_b_ref[...]
    b0 = bil_b_ref[0:1, 0:1]
    u03 = u0.reshape(1, 1, H)

    for c in range(NCH):
        xq = jnp.maximum(a_scr[pl.ds(c * CH, CH), :] * scq + shq, 0.0)
        dec_ref[pl.ds(c * CH, CH), :] = _dot(xq, ust, ((1,), (1,))) + bb
        sp_scr[pl.ds(c * PR, PR), :] = \
            jnp.sum(xq.reshape(PR, 128, H) * u03, axis=2) + b0
    scorep = sp_scr[...]

    row_iota = jax.lax.broadcasted_iota(jnp.int32, (NROWS, 128), 0)
    col_iota = jax.lax.broadcasted_iota(jnp.int32, (NROWS, 128), 1)
    validp = (row_iota * 128 + col_iota) < N
    SIGN = _i32(0x80000000)
    kp = jax.lax.bitcast_convert_type(scorep, jnp.int32)
    kp = jnp.where(kp < 0, kp ^ _i32(0x7FFFFFFF), kp)
    up = jnp.where(validp, kp ^ SIGN, _i32(0))

    bv1 = fcv_b_ref[pl.ds(0, 1), :]
    scv1, shv1 = mlp_layer1(b_scr, fcv_W_ref[0], bv1,
                            fcv_g_ref[pl.ds(0, 1), :],
                            fcv_be_ref[pl.ds(0, 1), :])

    prefix = jnp.zeros((1, 1), jnp.int32)
    remaining = jnp.full((1, 1), K, jnp.int32)
    for bit in range(30, -2, -2):
        mask_hi = _i32(~((1 << bit) - 1))
        masked = up & mask_hi
        c3 = jnp.sum((masked == (prefix | _i32(3 << bit)))
                     .astype(jnp.int32), axis=(0, 1)).reshape(1, 1)
        c2 = jnp.sum((masked == (prefix | _i32(2 << bit)))
                     .astype(jnp.int32), axis=(0, 1)).reshape(1, 1)
        c1 = jnp.sum((masked == (prefix | _i32(1 << bit)))
                     .astype(jnp.int32), axis=(0, 1)).reshape(1, 1)
        s32 = c3
        s21 = c3 + c2
        s10 = s21 + c1
        sel = jnp.where(
            s32 >= remaining, _i32(3),
            jnp.where(s21 >= remaining, _i32(2),
                      jnp.where(s10 >= remaining, _i32(1), _i32(0))))
        prefix = prefix | (sel << bit)
        remaining = remaining - jnp.where(
            sel == 3, _i32(0),
            jnp.where(sel == 2, s32, jnp.where(sel == 1, s21, s10)))

    t_signed = prefix ^ SIGN

    selp = validp & (kp >= t_signed)
    m = jnp.max(jnp.where(selp, scorep, -jnp.inf)).reshape(1, 1)

    scv, shv = mlp_layer2(b_scr, fcv_W_ref[1], fcv_b_ref[pl.ds(1, 1), :],
                          fcv_g_ref[pl.ds(1, 1), :],
                          fcv_be_ref[pl.ds(1, 1), :], scv1, shv1, bv1)

    pool = jnp.zeros((1, H), f32)
    zsum = jnp.zeros((1, 1), f32)
    for c in range(NCH):
        s_col = dec_ref[pl.ds(c * CH, CH), 0:1]
        kc = jax.lax.bitcast_convert_type(s_col, jnp.int32)
        kc = jnp.where(kc < 0, kc ^ _i32(0x7FFFFFFF), kc)
        niota = jax.lax.broadcasted_iota(jnp.int32, (CH, 1), 0) + c * CH
        sel = (niota < N) & (kc >= t_signed)
        w = jnp.where(sel, jnp.exp(s_col - m), 0.0)
        xv = jnp.maximum(b_scr[pl.ds(c * CH, CH), :] * scv + shv, 0.0)
        pool = pool + jnp.sum(w * xv, axis=0, keepdims=True)
        zsum = zsum + jnp.sum(w, axis=0, keepdims=True)
    pooled = pool / zsum

    logits_ref[...] = _dot(pooled, gfc_W_ref[...], ((1,), (1,))) \
        + gfc_b_ref[...]


def kernel(x, obs_x_idx, edge_index_01, edge_index_2, tf_Wv, tf_bv, tf_Wo,
           tf_bo, ln1_g, ln1_b, ff_W1, ff_b1, ff_W2, ff_b2, ln2_g, ln2_b,
           pool_w, pool_b, fcq_W, fcq_b, fcq_g, fcq_be, fcv_W, fcv_b, fcv_g,
           fcv_be, bil_W, bil_b, gfc_W, gfc_b):
    del edge_index_01, edge_index_2, pool_b
    idx = obs_x_idx.astype(jnp.int32)
    swap = lambda w: jnp.swapaxes(w, 1, 2)
    bfc = lambda w: jnp.swapaxes(w, 1, 2).astype(jnp.bfloat16)

    vmem = pl.BlockSpec(memory_space=pltpu.VMEM)
    operands = [
        x, swap(tf_Wv), tf_bv, swap(tf_Wo), tf_bo, ln1_g, ln1_b,
        swap(ff_W1), ff_b1, swap(ff_W2), ff_b2, ln2_g, ln2_b,
        pool_w.reshape(1, H),
        bfc(fcq_W), fcq_b, fcq_g, fcq_be, bfc(fcv_W), fcv_b, fcv_g, fcv_be,
        bil_W, bil_b.reshape(1, 2), gfc_W, gfc_b.reshape(1, 2),
    ]

    logits, dec = pl.pallas_call(
        _fused_kernel,
        grid_spec=pltpu.PrefetchScalarGridSpec(
            num_scalar_prefetch=1,
            grid=(),
            in_specs=[vmem] * len(operands),
            out_specs=[vmem, vmem],
            scratch_shapes=[
                pltpu.VMEM((NPAD, H), jnp.float32),
                pltpu.VMEM((NPAD, H), jnp.float32),
                pltpu.VMEM((OBS, H), jnp.float32),
                pltpu.VMEM((NROWS, 128), jnp.float32),
            ],
        ),
        out_shape=[
            jax.ShapeDtypeStruct((1, 2), jnp.float32),
            jax.ShapeDtypeStruct((NPAD, 2), jnp.float32),
        ],
    )(idx, *operands)
    return (logits, dec[:N])


# speedup vs baseline: 1.1592x; 1.0487x over previous
import jax
import jax.numpy as jnp
import numpy as np
from jax.experimental import pallas as pl
from jax.experimental.pallas import tpu as pltpu

H = 256
N = 10000
NPAD = 10240
NROWS = NPAD // 128
OBS = 128
NL = 2
K = 5000
EPS = 1e-5
CH = 2048
NCH = NPAD // CH
TAIL = N - (NCH - 1) * CH
PR = CH // 128

_PREC = jax.lax.Precision.DEFAULT


def _dot(a, b, dims=((1,), (0,))):
    return jax.lax.dot_general(a, b, dimension_numbers=(dims, ((), ())),
                               preferred_element_type=jnp.float32,
                               precision=_PREC)


def _i32(v):
    return jnp.int32(np.uint32(v & 0xFFFFFFFF).view(np.int32))


def _fused_kernel(idx_ref,
                  x_ref, tf_Wv_ref, tf_bv_ref, tf_Wo_ref, tf_bo_ref,
                  ln1_g_ref, ln1_b_ref, ff_W1_ref, ff_b1_ref, ff_W2_ref,
                  ff_b2_ref, ln2_g_ref, ln2_b_ref, pool_w_ref,
                  fcq_W_ref, fcq_b_ref, fcq_g_ref, fcq_be_ref,
                  fcv_W_ref, fcv_b_ref, fcv_g_ref, fcv_be_ref,
                  bil_W_ref, bil_b_ref, gfc_W_ref, gfc_b_ref,
                  logits_ref, dec_ref, a_scr, b_scr, h_scr, sp_scr):
    f32 = jnp.float32
    bf16 = jnp.bfloat16

    def gather_body(i, carry):
        ridx = idx_ref[i]
        h_scr[pl.ds(i, 1), :] = x_ref[pl.ds(ridx, 1), :]
        return carry

    jax.lax.fori_loop(0, OBS, gather_body, 0, unroll=8)
    h = h_scr[...]

    def _ln(t, g, b):
        mu = jnp.mean(t, axis=-1, keepdims=True)
        var = jnp.mean((t - mu) * (t - mu), axis=-1, keepdims=True)
        return (t - mu) * jax.lax.rsqrt(var + EPS) * g + b

    def obs_summarize(h):
        for l in range(NL):
            v = _dot(h, tf_Wv_ref[l]) + tf_bv_ref[pl.ds(l, 1), :]
            a = _dot(v, tf_Wo_ref[l]) + tf_bo_ref[pl.ds(l, 1), :]
            h = _ln(h + a, ln1_g_ref[pl.ds(l, 1), :],
                    ln1_b_ref[pl.ds(l, 1), :])
            ff = jnp.maximum(
                _dot(h, ff_W1_ref[l]) + ff_b1_ref[pl.ds(l, 1), :], 0.0)
            ff = _dot(ff, ff_W2_ref[l]) + ff_b2_ref[pl.ds(l, 1), :]
            h = _ln(h + ff, ln2_g_ref[pl.ds(l, 1), :],
                    ln2_b_ref[pl.ds(l, 1), :])
        h = jnp.maximum(h, 0.0)
        sc = _dot(h, pool_w_ref[...], ((1,), (1,)))
        sc = jnp.exp(sc - jnp.max(sc, axis=0, keepdims=True))
        sc = sc / jnp.sum(sc, axis=0, keepdims=True)
        return _dot(sc, h, ((0,), (0,)))

    npadf = float(NPAD - N)

    def chunk_rows(c):
        return TAIL if c == NCH - 1 else CH

    def mlp_layer1(dst_ref, W1, b1, g1, be1):
        s1 = jnp.zeros((1, H), f32)
        s2 = jnp.zeros((1, H), f32)
        for c in range(NCH):
            rows = chunk_rows(c)
            z = _dot(x_ref[pl.ds(c * CH, rows), :].astype(bf16), W1) + b1
            dst_ref[pl.ds(c * CH, rows), :] = z
            s1 = s1 + jnp.sum(z, axis=0, keepdims=True)
            s2 = s2 + jnp.sum(z * z, axis=0, keepdims=True)
        dst_ref[pl.ds(N, NPAD - N), :] = jnp.broadcast_to(b1, (NPAD - N, H))
        mu = s1 / float(N)
        var = s2 / float(N) - mu * mu
        sc1 = jax.lax.rsqrt(var + EPS) * g1
        return sc1, be1 - mu * sc1

    def mlp_layer2(dst_ref, W2, b2, g2, be2, sc1, sh1, b1):
        s1 = jnp.zeros((1, H), f32)
        s2 = jnp.zeros((1, H), f32)
        for c in range(NCH):
            y = jnp.maximum(dst_ref[pl.ds(c * CH, CH), :] * sc1 + sh1, 0.0)
            z = _dot(y.astype(bf16), W2) + b2
            dst_ref[pl.ds(c * CH, CH), :] = z
            s1 = s1 + jnp.sum(z, axis=0, keepdims=True)
            s2 = s2 + jnp.sum(z * z, axis=0, keepdims=True)
        y_pad1 = jnp.maximum(b1 * sc1 + sh1, 0.0)
        z_pad2 = _dot(y_pad1.astype(bf16), W2) + b2
        s1 = s1 - npadf * z_pad2
        s2 = s2 - npadf * z_pad2 * z_pad2
        mu = s1 / float(N)
        var = s2 / float(N) - mu * mu
        sc2 = jax.lax.rsqrt(var + EPS) * g2
        return sc2, be2 - mu * sc2

    bq1 = fcq_b_ref[pl.ds(0, 1), :]
    scq1, shq1 = mlp_layer1(a_scr, fcq_W_ref[0], bq1,
                            fcq_g_ref[pl.ds(0, 1), :],
                            fcq_be_ref[pl.ds(0, 1), :])
    obs_k = obs_summarize(h)
    u0 = _dot(obs_k, bil_W_ref[0])
    u1 = _dot(obs_k, bil_W_ref[1])
    scq, shq = mlp_layer2(a_scr, fcq_W_ref[1], fcq_b_ref[pl.ds(1, 1), :],
                          fcq_g_ref[pl.ds(1, 1), :],
                          fcq_be_ref[pl.ds(1, 1), :], scq1, shq1, bq1)

    ust = jnp.concatenate([u0, u1], axis=0)
    bb = bil_b_ref[...]
    b0 = bil_b_ref[0:1, 0:1]
    u03 = u0.reshape(1, 1, H)

    for c in range(NCH):
        xq = jnp.maximum(a_scr[pl.ds(c * CH, CH), :] * scq + shq, 0.0)
        dec_ref[pl.ds(c * CH, CH), :] = _dot(xq, ust, ((1,), (1,))) + bb
        sp_scr[pl.ds(c * PR, PR), :] = \
            jnp.sum(xq.reshape(PR, 128, H) * u03, axis=2) + b0
    scorep = sp_scr[...]

    row_iota = jax.lax.broadcasted_iota(jnp.int32, (NROWS, 128), 0)
    col_iota = jax.lax.broadcasted_iota(jnp.int32, (NROWS, 128), 1)
    validp = (row_iota * 128 + col_iota) < N
    SIGN = _i32(0x80000000)
    kp = jax.lax.bitcast_convert_type(scorep, jnp.int32)
    kp = jnp.where(kp < 0, kp ^ _i32(0x7FFFFFFF), kp)
    up = jnp.where(validp, kp ^ SIGN, _i32(0))

    bv1 = fcv_b_ref[pl.ds(0, 1), :]
    scv1, shv1 = mlp_layer1(b_scr, fcv_W_ref[0], bv1,
                            fcv_g_ref[pl.ds(0, 1), :],
                            fcv_be_ref[pl.ds(0, 1), :])

    prefix = jnp.zeros((1, 1), jnp.int32)
    remaining = jnp.full((1, 1), K, jnp.int32)
    for bit in range(30, -2, -2):
        mask_hi = _i32(~((1 << bit) - 1))
        masked = up & mask_hi
        c3 = jnp.sum((masked == (prefix | _i32(3 << bit)))
                     .astype(jnp.int32), axis=(0, 1)).reshape(1, 1)
        c2 = jnp.sum((masked == (prefix | _i32(2 << bit)))
                     .astype(jnp.int32), axis=(0, 1)).reshape(1, 1)
        c1 = jnp.sum((masked == (prefix | _i32(1 << bit)))
                     .astype(jnp.int32), axis=(0, 1)).reshape(1, 1)
        s32 = c3
        s21 = c3 + c2
        s10 = s21 + c1
        sel = jnp.where(
            s32 >= remaining, _i32(3),
            jnp.where(s21 >= remaining, _i32(2),
                      jnp.where(s10 >= remaining, _i32(1), _i32(0))))
        prefix = prefix | (sel << bit)
        remaining = remaining - jnp.where(
            sel == 3, _i32(0),
            jnp.where(sel == 2, s32, jnp.where(sel == 1, s21, s10)))

    t_signed = prefix ^ SIGN

    selp = validp & (kp >= t_signed)
    m = jnp.max(jnp.where(selp, scorep, -jnp.inf)).reshape(1, 1)

    scv, shv = mlp_layer2(b_scr, fcv_W_ref[1], fcv_b_ref[pl.ds(1, 1), :],
                          fcv_g_ref[pl.ds(1, 1), :],
                          fcv_be_ref[pl.ds(1, 1), :], scv1, shv1, bv1)

    pool = jnp.zeros((1, H), f32)
    zsum = jnp.zeros((1, 1), f32)
    for c in range(NCH):
        s_col = dec_ref[pl.ds(c * CH, CH), 0:1]
        kc = jax.lax.bitcast_convert_type(s_col, jnp.int32)
        kc = jnp.where(kc < 0, kc ^ _i32(0x7FFFFFFF), kc)
        niota = jax.lax.broadcasted_iota(jnp.int32, (CH, 1), 0) + c * CH
        sel = (niota < N) & (kc >= t_signed)
        w = jnp.where(sel, jnp.exp(s_col - m), 0.0)
        xv = jnp.maximum(b_scr[pl.ds(c * CH, CH), :] * scv + shv, 0.0)
        pool = pool + jnp.sum(w * xv, axis=0, keepdims=True)
        zsum = zsum + jnp.sum(w, axis=0, keepdims=True)
    pooled = pool / zsum

    logits_ref[...] = _dot(pooled, gfc_W_ref[...], ((1,), (1,))) \
        + gfc_b_ref[...]


def kernel(x, obs_x_idx, edge_index_01, edge_index_2, tf_Wv, tf_bv, tf_Wo,
           tf_bo, ln1_g, ln1_b, ff_W1, ff_b1, ff_W2, ff_b2, ln2_g, ln2_b,
           pool_w, pool_b, fcq_W, fcq_b, fcq_g, fcq_be, fcv_W, fcv_b, fcv_g,
           fcv_be, bil_W, bil_b, gfc_W, gfc_b):
    del edge_index_01, edge_index_2, pool_b
    idx = obs_x_idx.astype(jnp.int32)
    swap = lambda w: jnp.swapaxes(w, 1, 2)
    bfc = lambda w: jnp.swapaxes(w, 1, 2).astype(jnp.bfloat16)

    vmem = pl.BlockSpec(memory_space=pltpu.VMEM)
    operands = [
        x, swap(tf_Wv), tf_bv, swap(tf_Wo), tf_bo, ln1_g, ln1_b,
        swap(ff_W1), ff_b1, swap(ff_W2), ff_b2, ln2_g, ln2_b,
        pool_w.reshape(1, H),
        bfc(fcq_W), fcq_b, fcq_g, fcq_be, bfc(fcv_W), fcv_b, fcv_g, fcv_be,
        bil_W, bil_b.reshape(1, 2), gfc_W, gfc_b.reshape(1, 2),
    ]

    logits, dec = pl.pallas_call(
        _fused_kernel,
        grid_spec=pltpu.PrefetchScalarGridSpec(
            num_scalar_prefetch=1,
            grid=(),
            in_specs=[vmem] * len(operands),
            out_specs=[vmem, vmem],
            scratch_shapes=[
                pltpu.VMEM((NPAD, H), jnp.float32),
                pltpu.VMEM((NPAD, H), jnp.float32),
                pltpu.VMEM((OBS, H), jnp.float32),
                pltpu.VMEM((NROWS, 128), jnp.float32),
            ],
        ),
        out_shape=[
            jax.ShapeDtypeStruct((1, 2), jnp.float32),
            jax.ShapeDtypeStruct((NPAD, 2), jnp.float32),
        ],
    )(idx, *operands)
    return (logits, dec[:N])


# speedup vs baseline: 1.5706x; 1.3550x over previous
import jax
import jax.numpy as jnp
import numpy as np
from jax.experimental import pallas as pl
from jax.experimental.pallas import tpu as pltpu

H = 256
N = 10000
NPAD = 10240
NROWS = NPAD // 128
OBS = 128
NL = 2
K = 5000
EPS = 1e-5
CH = 2048
NCH = NPAD // CH
TAIL = N - (NCH - 1) * CH
PR = CH // 128

_PREC = jax.lax.Precision.DEFAULT


def _dot(a, b, dims=((1,), (0,))):
    return jax.lax.dot_general(a, b, dimension_numbers=(dims, ((), ())),
                               preferred_element_type=jnp.float32,
                               precision=_PREC)


def _i32(v):
    return jnp.int32(np.uint32(v & 0xFFFFFFFF).view(np.int32))


def _fused_kernel(idx_ref,
                  x_ref, tf_Wv_ref, tf_bv_ref, tf_Wo_ref, tf_bo_ref,
                  ln1_g_ref, ln1_b_ref, ff_W1_ref, ff_b1_ref, ff_W2_ref,
                  ff_b2_ref, ln2_g_ref, ln2_b_ref, pool_w_ref,
                  fcq_W_ref, fcq_b_ref, fcq_g_ref, fcq_be_ref,
                  fcv_W_ref, fcv_b_ref, fcv_g_ref, fcv_be_ref,
                  bil_W_ref, bil_b_ref, gfc_W_ref, gfc_b_ref,
                  logits_ref, dec_ref, a_scr, b_scr, h_scr, sp_scr):
    f32 = jnp.float32
    bf16 = jnp.bfloat16

    def gather_body(i, carry):
        ridx = idx_ref[i]
        h_scr[pl.ds(i, 1), :] = x_ref[pl.ds(ridx, 1), :]
        return carry

    jax.lax.fori_loop(0, OBS, gather_body, 0, unroll=8)
    h = h_scr[...]

    def _ln(t, g, b):
        mu = jnp.mean(t, axis=-1, keepdims=True)
        var = jnp.mean((t - mu) * (t - mu), axis=-1, keepdims=True)
        return (t - mu) * jax.lax.rsqrt(var + EPS) * g + b

    def obs_summarize(h):
        for l in range(NL):
            v = _dot(h, tf_Wv_ref[l], ((1,), (1,))) + tf_bv_ref[pl.ds(l, 1), :]
            a = _dot(v, tf_Wo_ref[l], ((1,), (1,))) + tf_bo_ref[pl.ds(l, 1), :]
            h = _ln(h + a, ln1_g_ref[pl.ds(l, 1), :],
                    ln1_b_ref[pl.ds(l, 1), :])
            ff = jnp.maximum(
                _dot(h, ff_W1_ref[l], ((1,), (1,))) + ff_b1_ref[pl.ds(l, 1), :], 0.0)
            ff = _dot(ff, ff_W2_ref[l], ((1,), (1,))) + ff_b2_ref[pl.ds(l, 1), :]
            h = _ln(h + ff, ln2_g_ref[pl.ds(l, 1), :],
                    ln2_b_ref[pl.ds(l, 1), :])
        h = jnp.maximum(h, 0.0)
        sc = _dot(h, pool_w_ref[...], ((1,), (1,)))
        sc = jnp.exp(sc - jnp.max(sc, axis=0, keepdims=True))
        sc = sc / jnp.sum(sc, axis=0, keepdims=True)
        return _dot(sc, h, ((0,), (0,)))

    npadf = float(NPAD - N)

    def chunk_rows(c):
        return TAIL if c == NCH - 1 else CH

    def mlp_layer1(dst_ref, W1, b1, g1, be1):
        W1 = W1.astype(bf16)
        s1 = jnp.zeros((1, H), f32)
        s2 = jnp.zeros((1, H), f32)
        for c in range(NCH):
            rows = chunk_rows(c)
            z = _dot(x_ref[pl.ds(c * CH, rows), :].astype(bf16), W1,
                     ((1,), (1,))) + b1
            dst_ref[pl.ds(c * CH, rows), :] = z
            s1 = s1 + jnp.sum(z, axis=0, keepdims=True)
            s2 = s2 + jnp.sum(z * z, axis=0, keepdims=True)
        dst_ref[pl.ds(N, NPAD - N), :] = jnp.broadcast_to(b1, (NPAD - N, H))
        mu = s1 / float(N)
        var = s2 / float(N) - mu * mu
        sc1 = jax.lax.rsqrt(var + EPS) * g1
        return sc1, be1 - mu * sc1

    def mlp_layer2(dst_ref, W2, b2, g2, be2, sc1, sh1, b1):
        W2 = W2.astype(bf16)
        s1 = jnp.zeros((1, H), f32)
        s2 = jnp.zeros((1, H), f32)
        for c in range(NCH):
            y = jnp.maximum(dst_ref[pl.ds(c * CH, CH), :] * sc1 + sh1, 0.0)
            z = _dot(y.astype(bf16), W2, ((1,), (1,))) + b2
            dst_ref[pl.ds(c * CH, CH), :] = z
            s1 = s1 + jnp.sum(z, axis=0, keepdims=True)
            s2 = s2 + jnp.sum(z * z, axis=0, keepdims=True)
        y_pad1 = jnp.maximum(b1 * sc1 + sh1, 0.0)
        z_pad2 = _dot(y_pad1.astype(bf16), W2, ((1,), (1,))) + b2
        s1 = s1 - npadf * z_pad2
        s2 = s2 - npadf * z_pad2 * z_pad2
        mu = s1 / float(N)
        var = s2 / float(N) - mu * mu
        sc2 = jax.lax.rsqrt(var + EPS) * g2
        return sc2, be2 - mu * sc2

    bq1 = fcq_b_ref[pl.ds(0, 1), :]
    scq1, shq1 = mlp_layer1(a_scr, fcq_W_ref[0], bq1,
                            fcq_g_ref[pl.ds(0, 1), :],
                            fcq_be_ref[pl.ds(0, 1), :])
    obs_k = obs_summarize(h)
    u0 = _dot(obs_k, bil_W_ref[0])
    u1 = _dot(obs_k, bil_W_ref[1])
    scq, shq = mlp_layer2(a_scr, fcq_W_ref[1], fcq_b_ref[pl.ds(1, 1), :],
                          fcq_g_ref[pl.ds(1, 1), :],
                          fcq_be_ref[pl.ds(1, 1), :], scq1, shq1, bq1)

    ust = jnp.concatenate([u0, u1], axis=0)
    bb = bil_b_ref[...]
    b0 = bil_b_ref[0:1, 0:1]
    u03 = u0.reshape(1, 1, H)

    for c in range(NCH - 1):
        xq = jnp.maximum(a_scr[pl.ds(c * CH, CH), :] * scq + shq, 0.0)
        dec_ref[pl.ds(c * CH, CH), :] = _dot(xq, ust, ((1,), (1,))) + bb
        sp_scr[pl.ds(c * PR, PR), :] = \
            jnp.sum(xq.reshape(PR, 128, H) * u03, axis=2) + b0
    xqt = jnp.maximum(
        a_scr[pl.ds((NCH - 1) * CH, TAIL), :] * scq + shq, 0.0)
    dec_ref[pl.ds((NCH - 1) * CH, TAIL), :] = \
        _dot(xqt, ust, ((1,), (1,))) + bb
    SPDONE = ((NCH - 1) * PR // 2) * 2 * 128
    xqs = jnp.maximum(
        a_scr[pl.ds(SPDONE, NPAD - SPDONE), :] * scq + shq, 0.0)
    sp_scr[pl.ds(SPDONE // 128, (NPAD - SPDONE) // 128), :] = \
        jnp.sum(xqs.reshape((NPAD - SPDONE) // 128, 128, H) * u03,
                axis=2) + b0
    scorep = sp_scr[...]

    row_iota = jax.lax.broadcasted_iota(jnp.int32, (NROWS, 128), 0)
    col_iota = jax.lax.broadcasted_iota(jnp.int32, (NROWS, 128), 1)
    validp = (row_iota * 128 + col_iota) < N
    SIGN = _i32(0x80000000)
    kp = jax.lax.bitcast_convert_type(scorep, jnp.int32)
    kp = jnp.where(kp < 0, kp ^ _i32(0x7FFFFFFF), kp)
    up = jnp.where(validp, kp ^ SIGN, _i32(0))

    bv1 = fcv_b_ref[pl.ds(0, 1), :]
    scv1, shv1 = mlp_layer1(b_scr, fcv_W_ref[0], bv1,
                            fcv_g_ref[pl.ds(0, 1), :],
                            fcv_be_ref[pl.ds(0, 1), :])

    prefix = jnp.zeros((1, 1), jnp.int32)
    remaining = jnp.full((1, 1), K, jnp.int32)
    for bit in range(30, -2, -2):
        mask_hi = _i32(~((1 << bit) - 1))
        masked = up & mask_hi
        c3 = jnp.sum((masked == (prefix | _i32(3 << bit)))
                     .astype(jnp.int32), axis=(0, 1)).reshape(1, 1)
        c2 = jnp.sum((masked == (prefix | _i32(2 << bit)))
                     .astype(jnp.int32), axis=(0, 1)).reshape(1, 1)
        c1 = jnp.sum((masked == (prefix | _i32(1 << bit)))
                     .astype(jnp.int32), axis=(0, 1)).reshape(1, 1)
        s32 = c3
        s21 = c3 + c2
        s10 = s21 + c1
        sel = jnp.where(
            s32 >= remaining, _i32(3),
            jnp.where(s21 >= remaining, _i32(2),
                      jnp.where(s10 >= remaining, _i32(1), _i32(0))))
        prefix = prefix | (sel << bit)
        remaining = remaining - jnp.where(
            sel == 3, _i32(0),
            jnp.where(sel == 2, s32, jnp.where(sel == 1, s21, s10)))

    t_signed = prefix ^ SIGN

    selp = validp & (kp >= t_signed)
    m = jnp.max(jnp.where(selp, scorep, -jnp.inf)).reshape(1, 1)

    scv, shv = mlp_layer2(b_scr, fcv_W_ref[1], fcv_b_ref[pl.ds(1, 1), :],
                          fcv_g_ref[pl.ds(1, 1), :],
                          fcv_be_ref[pl.ds(1, 1), :], scv1, shv1, bv1)

    pool = jnp.zeros((1, H), f32)
    zsum = jnp.zeros((1, 1), f32)
    for c in range(NCH):
        rows = chunk_rows(c)
        s_col = dec_ref[pl.ds(c * CH, rows), 0:1]
        kc = jax.lax.bitcast_convert_type(s_col, jnp.int32)
        kc = jnp.where(kc < 0, kc ^ _i32(0x7FFFFFFF), kc)
        sel = kc >= t_signed
        w = jnp.where(sel, jnp.exp(s_col - m), 0.0)
        xv = jnp.maximum(b_scr[pl.ds(c * CH, rows), :] * scv + shv, 0.0)
        pool = pool + jnp.sum(w * xv, axis=0, keepdims=True)
        zsum = zsum + jnp.sum(w, axis=0, keepdims=True)
    pooled = pool / zsum

    logits_ref[...] = _dot(pooled, gfc_W_ref[...], ((1,), (1,))) \
        + gfc_b_ref[...]


def kernel(x, obs_x_idx, edge_index_01, edge_index_2, tf_Wv, tf_bv, tf_Wo,
           tf_bo, ln1_g, ln1_b, ff_W1, ff_b1, ff_W2, ff_b2, ln2_g, ln2_b,
           pool_w, pool_b, fcq_W, fcq_b, fcq_g, fcq_be, fcv_W, fcv_b, fcv_g,
           fcv_be, bil_W, bil_b, gfc_W, gfc_b):
    del edge_index_01, edge_index_2, pool_b
    idx = obs_x_idx.astype(jnp.int32)

    vmem = pl.BlockSpec(memory_space=pltpu.VMEM)
    operands = [
        x, tf_Wv, tf_bv, tf_Wo, tf_bo, ln1_g, ln1_b,
        ff_W1, ff_b1, ff_W2, ff_b2, ln2_g, ln2_b,
        pool_w.reshape(1, H),
        fcq_W, fcq_b, fcq_g, fcq_be, fcv_W, fcv_b, fcv_g, fcv_be,
        bil_W, bil_b.reshape(1, 2), gfc_W, gfc_b.reshape(1, 2),
    ]

    logits, dec = pl.pallas_call(
        _fused_kernel,
        grid_spec=pltpu.PrefetchScalarGridSpec(
            num_scalar_prefetch=1,
            grid=(),
            in_specs=[vmem] * len(operands),
            out_specs=[vmem, vmem],
            scratch_shapes=[
                pltpu.VMEM((NPAD, H), jnp.float32),
                pltpu.VMEM((NPAD, H), jnp.float32),
                pltpu.VMEM((OBS, H), jnp.float32),
                pltpu.VMEM((NROWS, 128), jnp.float32),
            ],
        ),
        out_shape=[
            jax.ShapeDtypeStruct((1, 2), jnp.float32),
            jax.ShapeDtypeStruct((N, 2), jnp.float32),
        ],
    )(idx, *operands)
    return (logits, dec)


# speedup vs baseline: 1.5916x; 1.0134x over previous
import jax
import jax.numpy as jnp
import numpy as np
from jax.experimental import pallas as pl
from jax.experimental.pallas import tpu as pltpu

H = 256
N = 10000
NPAD = 10240
NROWS = NPAD // 128
OBS = 128
NL = 2
K = 5000
EPS = 1e-5
CH = 2048
NCH = NPAD // CH
TAIL = N - (NCH - 1) * CH
PR = CH // 128

_PREC = jax.lax.Precision.DEFAULT


def _dot(a, b, dims=((1,), (0,))):
    return jax.lax.dot_general(a, b, dimension_numbers=(dims, ((), ())),
                               preferred_element_type=jnp.float32,
                               precision=_PREC)


def _i32(v):
    return jnp.int32(np.uint32(v & 0xFFFFFFFF).view(np.int32))


def _fused_kernel(idx_ref,
                  x_ref, tf_Wv_ref, tf_bv_ref, tf_Wo_ref, tf_bo_ref,
                  ln1_g_ref, ln1_b_ref, ff_W1_ref, ff_b1_ref, ff_W2_ref,
                  ff_b2_ref, ln2_g_ref, ln2_b_ref, pool_w_ref,
                  fcq_W_ref, fcq_b_ref, fcq_g_ref, fcq_be_ref,
                  fcv_W_ref, fcv_b_ref, fcv_g_ref, fcv_be_ref,
                  bil_W_ref, bil_b_ref, gfc_W_ref, gfc_b_ref,
                  logits_ref, dec_ref, a_scr, b_scr, h_scr, sp_scr):
    f32 = jnp.float32
    bf16 = jnp.bfloat16

    def gather_rows():
        for i in range(OBS):
            h_scr[pl.ds(i, 1), :] = x_ref[pl.ds(idx_ref[i], 1), :]

    def _ln(t, g, b):
        mu = jnp.mean(t, axis=-1, keepdims=True)
        var = jnp.mean((t - mu) * (t - mu), axis=-1, keepdims=True)
        return (t - mu) * jax.lax.rsqrt(var + EPS) * g + b

    def obs_summarize(h):
        for l in range(NL):
            v = _dot(h, tf_Wv_ref[l], ((1,), (1,))) + tf_bv_ref[pl.ds(l, 1), :]
            a = _dot(v, tf_Wo_ref[l], ((1,), (1,))) + tf_bo_ref[pl.ds(l, 1), :]
            h = _ln(h + a, ln1_g_ref[pl.ds(l, 1), :],
                    ln1_b_ref[pl.ds(l, 1), :])
            ff = jnp.maximum(
                _dot(h, ff_W1_ref[l], ((1,), (1,))) + ff_b1_ref[pl.ds(l, 1), :], 0.0)
            ff = _dot(ff, ff_W2_ref[l], ((1,), (1,))) + ff_b2_ref[pl.ds(l, 1), :]
            h = _ln(h + ff, ln2_g_ref[pl.ds(l, 1), :],
                    ln2_b_ref[pl.ds(l, 1), :])
        h = jnp.maximum(h, 0.0)
        sc = _dot(h, pool_w_ref[...], ((1,), (1,)))
        sc = jnp.exp(sc - jnp.max(sc, axis=0, keepdims=True))
        sc = sc / jnp.sum(sc, axis=0, keepdims=True)
        return _dot(sc, h, ((0,), (0,)))

    npadf = float(NPAD - N)

    def chunk_rows(c):
        return TAIL if c == NCH - 1 else CH

    def mlp_layer1(dst_ref, W1, b1, g1, be1):
        W1 = W1.astype(bf16)
        s1 = jnp.zeros((1, H), f32)
        s2 = jnp.zeros((1, H), f32)
        for c in range(NCH):
            rows = chunk_rows(c)
            z = _dot(x_ref[pl.ds(c * CH, rows), :].astype(bf16), W1,
                     ((1,), (1,))) + b1
            dst_ref[pl.ds(c * CH, rows), :] = z
            s1 = s1 + jnp.sum(z, axis=0, keepdims=True)
            s2 = s2 + jnp.sum(z * z, axis=0, keepdims=True)
        dst_ref[pl.ds(N, NPAD - N), :] = jnp.broadcast_to(b1, (NPAD - N, H))
        mu = s1 / float(N)
        var = s2 / float(N) - mu * mu
        sc1 = jax.lax.rsqrt(var + EPS) * g1
        return sc1, be1 - mu * sc1

    def mlp_layer2(dst_ref, W2, b2, g2, be2, sc1, sh1, b1):
        W2 = W2.astype(bf16)
        s1 = jnp.zeros((1, H), f32)
        s2 = jnp.zeros((1, H), f32)
        for c in range(NCH):
            y = jnp.maximum(dst_ref[pl.ds(c * CH, CH), :] * sc1 + sh1, 0.0)
            z = _dot(y.astype(bf16), W2, ((1,), (1,))) + b2
            dst_ref[pl.ds(c * CH, CH), :] = z
            s1 = s1 + jnp.sum(z, axis=0, keepdims=True)
            s2 = s2 + jnp.sum(z * z, axis=0, keepdims=True)
        y_pad1 = jnp.maximum(b1 * sc1 + sh1, 0.0)
        z_pad2 = _dot(y_pad1.astype(bf16), W2, ((1,), (1,))) + b2
        s1 = s1 - npadf * z_pad2
        s2 = s2 - npadf * z_pad2 * z_pad2
        mu = s1 / float(N)
        var = s2 / float(N) - mu * mu
        sc2 = jax.lax.rsqrt(var + EPS) * g2
        return sc2, be2 - mu * sc2

    bq1 = fcq_b_ref[pl.ds(0, 1), :]
    gather_rows()
    scq1, shq1 = mlp_layer1(a_scr, fcq_W_ref[0], bq1,
                            fcq_g_ref[pl.ds(0, 1), :],
                            fcq_be_ref[pl.ds(0, 1), :])
    obs_k = obs_summarize(h_scr[...])
    u0 = _dot(obs_k, bil_W_ref[0])
    u1 = _dot(obs_k, bil_W_ref[1])
    scq, shq = mlp_layer2(a_scr, fcq_W_ref[1], fcq_b_ref[pl.ds(1, 1), :],
                          fcq_g_ref[pl.ds(1, 1), :],
                          fcq_be_ref[pl.ds(1, 1), :], scq1, shq1, bq1)

    ust = jnp.concatenate([u0, u1], axis=0)
    bb = bil_b_ref[...]
    b0 = bil_b_ref[0:1, 0:1]
    u03 = u0.reshape(1, 1, H)

    for c in range(NCH - 1):
        xq = jnp.maximum(a_scr[pl.ds(c * CH, CH), :] * scq + shq, 0.0)
        dec_ref[pl.ds(c * CH, CH), :] = _dot(xq, ust, ((1,), (1,))) + bb
        sp_scr[pl.ds(c * PR, PR), :] = \
            jnp.sum(xq.reshape(PR, 128, H) * u03, axis=2) + b0
    xqt = jnp.maximum(
        a_scr[pl.ds((NCH - 1) * CH, TAIL), :] * scq + shq, 0.0)
    dec_ref[pl.ds((NCH - 1) * CH, TAIL), :] = \
        _dot(xqt, ust, ((1,), (1,))) + bb
    SPDONE = ((NCH - 1) * PR // 2) * 2 * 128
    xqs = jnp.maximum(
        a_scr[pl.ds(SPDONE, NPAD - SPDONE), :] * scq + shq, 0.0)
    sp_scr[pl.ds(SPDONE // 128, (NPAD - SPDONE) // 128), :] = \
        jnp.sum(xqs.reshape((NPAD - SPDONE) // 128, 128, H) * u03,
                axis=2) + b0
    scorep = sp_scr[...]

    row_iota = jax.lax.broadcasted_iota(jnp.int32, (NROWS, 128), 0)
    col_iota = jax.lax.broadcasted_iota(jnp.int32, (NROWS, 128), 1)
    validp = (row_iota * 128 + col_iota) < N
    SIGN = _i32(0x80000000)
    kp = jax.lax.bitcast_convert_type(scorep, jnp.int32)
    kp = jnp.where(kp < 0, kp ^ _i32(0x7FFFFFFF), kp)
    up = jnp.where(validp, kp ^ SIGN, _i32(0))

    bv1 = fcv_b_ref[pl.ds(0, 1), :]
    scv1, shv1 = mlp_layer1(b_scr, fcv_W_ref[0], bv1,
                            fcv_g_ref[pl.ds(0, 1), :],
                            fcv_be_ref[pl.ds(0, 1), :])

    prefix = jnp.zeros((1, 1), jnp.int32)
    remaining = jnp.full((1, 1), K, jnp.int32)
    for bit in range(30, -2, -2):
        mask_hi = _i32(~((1 << bit) - 1))
        masked = up & mask_hi
        c3 = jnp.sum((masked == (prefix | _i32(3 << bit)))
                     .astype(jnp.int32), axis=(0, 1)).reshape(1, 1)
        c2 = jnp.sum((masked == (prefix | _i32(2 << bit)))
                     .astype(jnp.int32), axis=(0, 1)).reshape(1, 1)
        c1 = jnp.sum((masked == (prefix | _i32(1 << bit)))
                     .astype(jnp.int32), axis=(0, 1)).reshape(1, 1)
        s32 = c3
        s21 = c3 + c2
        s10 = s21 + c1
        sel = jnp.where(
            s32 >= remaining, _i32(3),
            jnp.where(s21 >= remaining, _i32(2),
                      jnp.where(s10 >= remaining, _i32(1), _i32(0))))
        prefix = prefix | (sel << bit)
        remaining = remaining - jnp.where(
            sel == 3, _i32(0),
            jnp.where(sel == 2, s32, jnp.where(sel == 1, s21, s10)))

    t_signed = prefix ^ SIGN

    selp = validp & (kp >= t_signed)
    m = jnp.max(jnp.where(selp, scorep, -jnp.inf)).reshape(1, 1)

    scv, shv = mlp_layer2(b_scr, fcv_W_ref[1], fcv_b_ref[pl.ds(1, 1), :],
                          fcv_g_ref[pl.ds(1, 1), :],
                          fcv_be_ref[pl.ds(1, 1), :], scv1, shv1, bv1)

    pool = jnp.zeros((1, H), f32)
    zsum = jnp.zeros((1, 1), f32)
    for c in range(NCH):
        rows = chunk_rows(c)
        s_col = dec_ref[pl.ds(c * CH, rows), 0:1]
        kc = jax.lax.bitcast_convert_type(s_col, jnp.int32)
        kc = jnp.where(kc < 0, kc ^ _i32(0x7FFFFFFF), kc)
        sel = kc >= t_signed
        w = jnp.where(sel, jnp.exp(s_col - m), 0.0)
        xv = jnp.maximum(b_scr[pl.ds(c * CH, rows), :] * scv + shv, 0.0)
        pool = pool + jnp.sum(w * xv, axis=0, keepdims=True)
        zsum = zsum + jnp.sum(w, axis=0, keepdims=True)
    pooled = pool / zsum

    logits_ref[...] = _dot(pooled, gfc_W_ref[...], ((1,), (1,))) \
        + gfc_b_ref[...]


def kernel(x, obs_x_idx, edge_index_01, edge_index_2, tf_Wv, tf_bv, tf_Wo,
           tf_bo, ln1_g, ln1_b, ff_W1, ff_b1, ff_W2, ff_b2, ln2_g, ln2_b,
           pool_w, pool_b, fcq_W, fcq_b, fcq_g, fcq_be, fcv_W, fcv_b, fcv_g,
           fcv_be, bil_W, bil_b, gfc_W, gfc_b):
    del edge_index_01, edge_index_2, pool_b
    idx = obs_x_idx.astype(jnp.int32)

    vmem = pl.BlockSpec(memory_space=pltpu.VMEM)
    operands = [
        x, tf_Wv, tf_bv, tf_Wo, tf_bo, ln1_g, ln1_b,
        ff_W1, ff_b1, ff_W2, ff_b2, ln2_g, ln2_b,
        pool_w.reshape(1, H),
        fcq_W, fcq_b, fcq_g, fcq_be, fcv_W, fcv_b, fcv_g, fcv_be,
        bil_W, bil_b.reshape(1, 2), gfc_W, gfc_b.reshape(1, 2),
    ]

    logits, dec = pl.pallas_call(
        _fused_kernel,
        grid_spec=pltpu.PrefetchScalarGridSpec(
            num_scalar_prefetch=1,
            grid=(),
            in_specs=[vmem] * len(operands),
            out_specs=[vmem, vmem],
            scratch_shapes=[
                pltpu.VMEM((NPAD, H), jnp.float32),
                pltpu.VMEM((NPAD, H), jnp.float32),
                pltpu.VMEM((OBS, H), jnp.float32),
                pltpu.VMEM((NROWS, 128), jnp.float32),
            ],
        ),
        out_shape=[
            jax.ShapeDtypeStruct((1, 2), jnp.float32),
            jax.ShapeDtypeStruct((N, 2), jnp.float32),
        ],
    )(idx, *operands)
    return (logits, dec)
